# baseline (speedup 1.0000x reference)
import functools

import jax
import jax.numpy as jnp
from jax import lax
from jax.experimental import pallas as pl
from jax.experimental.pallas import tpu as pltpu
from jax.experimental.pallas import tpu_sc as plsc

F = 26
B = 4096
L = 20
V = 100000
D = 16

NC = 2
NS = 16
NW = NC * NS

NB = 16
NF = 2
FPW = F // NF
R = B // NB
IDX_PER_CHUNK = R * L
GROWS = 128
NWIN = 2
HBAGS = R // NWIN
VQ = V // 8

_mesh = plsc.VectorSubcoreMesh(
    core_axis_name="c", subcore_axis_name="s", num_cores=NC, num_subcores=NS
)


def _tc_prep_body(tab_ref, idx_ref, ptab_ref, pidx_ref):
    eye = jnp.eye(D, dtype=jnp.float32)
    for vv in range(8):
        x = tab_ref[0, :, vv * VQ:(vv + 1) * VQ]
        ptab_ref[0, :, vv * D:(vv + 1) * D] = jax.lax.dot_general(
            x, eye, (((0,), (0,)), ((), ())),
            preferred_element_type=jnp.float32,
        )
    f = pl.program_id(0)
    v = idx_ref[0]
    q = (v >= VQ).astype(jnp.int32)
    for k in range(2, 8):
        q = q + (v >= k * VQ).astype(jnp.int32)
    pidx_ref[0] = v * 8 - q * (8 * VQ - 1) + f * V


def _tc_prep(tab_t, idx_t):
    return pl.pallas_call(
        _tc_prep_body,
        grid=(F,),
        in_specs=[
            pl.BlockSpec((1, D, V), lambda f: (f, 0, 0)),
            pl.BlockSpec((1, L, B), lambda f: (f, 0, 0)),
        ],
        out_specs=[
            pl.BlockSpec((1, VQ, 128), lambda f: (f, 0, 0)),
            pl.BlockSpec((1, L, B), lambda f: (f, 0, 0)),
        ],
        out_shape=[
            jax.ShapeDtypeStruct((F, VQ, 128), jnp.float32),
            jax.ShapeDtypeStruct((F, L, B), jnp.int32),
        ],
        compiler_params=pltpu.CompilerParams(
            fuse_transposed_lhs_in_matmul=True,
            dimension_semantics=("arbitrary",),
        ),
    )(tab_t, idx_t)


@functools.partial(
    pl.kernel,
    out_type=jax.ShapeDtypeStruct((B, F * D), jnp.float32),
    mesh=_mesh,
    scratch_types=[
        pltpu.VMEM((L, R), jnp.int32),
        pltpu.VMEM((IDX_PER_CHUNK, D), jnp.float32),
        pltpu.VMEM((R, D), jnp.float32),
        pltpu.SemaphoreType.DMA((NWIN,)),
    ],
    compiler_params=pltpu.CompilerParams(use_tc_tiling_on_sc=False),
)
def _pooled_gather(tab_hbm, idxt_hbm, out_hbm, idx_v, g_v, o_v, sems):
    wid = lax.axis_index("s") * NC + lax.axis_index("c")
    fh = wid // NB
    bc = wid - fh * NB
    bag0 = pl.multiple_of(bc * R, 8)

    @pl.loop(0, FPW)
    def _feat(fi):
        f = fh * FPW + fi
        pltpu.sync_copy(idxt_hbm.at[f, :, pl.ds(bag0, R)], idx_v)
        copies = [
            pltpu.async_copy(
                tab_hbm.at[idx_v.at[l, pl.ds(h * GROWS, GROWS)]],
                g_v.at[pl.ds(l * R + h * GROWS, GROWS)],
                sems.at[h],
            )
            for h in range(NWIN)
            for l in range(L)
        ]
        for h in range(NWIN):
            for cp in copies[h * L : (h + 1) * L]:
                cp.wait()

            @pl.loop(0, HBAGS)
            def _bag(b):
                base = h * GROWS + b
                acc = g_v[base, :]
                for l in range(1, L):
                    acc = acc + g_v[l * R + base, :]
                o_v[base, :] = acc

        pltpu.sync_copy(o_v, out_hbm.at[pl.ds(bag0, R), pl.ds(f * D, D)])


def kernel(indices, tables):
    packed, gidx = _tc_prep(
        tables.transpose(0, 2, 1), indices.transpose(0, 2, 1)
    )
    return _pooled_gather(packed.reshape(F * V, D), gidx)


# speedup vs baseline: 1.0334x; 1.0334x over previous
import functools

import jax
import jax.numpy as jnp
from jax import lax
from jax.experimental import pallas as pl
from jax.experimental.pallas import tpu as pltpu
from jax.experimental.pallas import tpu_sc as plsc

F = 26
B = 4096
L = 20
V = 100000
D = 16

NC = 2
NS = 16
NW = NC * NS

NB = 16
NF = 2
FPW = F // NF
R = B // NB
IDX_PER_CHUNK = R * L
GROWS = 128
NWIN = 2
HBAGS = R // NWIN
VQ = V // 8

_mesh = plsc.VectorSubcoreMesh(
    core_axis_name="c", subcore_axis_name="s", num_cores=NC, num_subcores=NS
)


def _tc_prep_body(tab_ref, idx_ref, ptab_ref, pidx_ref):
    for vv in range(8):
        x = tab_ref[0, :, vv * VQ:(vv + 1) * VQ]
        ptab_ref[0, :, vv * D:(vv + 1) * D] = x.T
    f = pl.program_id(0)
    v = idx_ref[0]
    q = (v >= VQ).astype(jnp.int32)
    for k in range(2, 8):
        q = q + (v >= k * VQ).astype(jnp.int32)
    pidx_ref[0] = v * 8 - q * (8 * VQ - 1) + f * V


def _tc_prep(tab_t, idx_t):
    return pl.pallas_call(
        _tc_prep_body,
        grid=(F,),
        in_specs=[
            pl.BlockSpec((1, D, V), lambda f: (f, 0, 0)),
            pl.BlockSpec((1, L, B), lambda f: (f, 0, 0)),
        ],
        out_specs=[
            pl.BlockSpec((1, VQ, 128), lambda f: (f, 0, 0)),
            pl.BlockSpec((1, L, B), lambda f: (f, 0, 0)),
        ],
        out_shape=[
            jax.ShapeDtypeStruct((F, VQ, 128), jnp.float32),
            jax.ShapeDtypeStruct((F, L, B), jnp.int32),
        ],
    )(tab_t, idx_t)


@functools.partial(
    pl.kernel,
    out_type=jax.ShapeDtypeStruct((B, F * D), jnp.float32),
    mesh=_mesh,
    scratch_types=[
        pltpu.VMEM((L, R), jnp.int32),
        pltpu.VMEM((IDX_PER_CHUNK, D), jnp.float32),
        pltpu.VMEM((R, D), jnp.float32),
        pltpu.SemaphoreType.DMA((NWIN,)),
    ],
    compiler_params=pltpu.CompilerParams(use_tc_tiling_on_sc=False),
)
def _pooled_gather(tab_hbm, idxt_hbm, out_hbm, idx_v, g_v, o_v, sems):
    wid = lax.axis_index("s") * NC + lax.axis_index("c")
    fh = wid // NB
    bc = wid - fh * NB
    bag0 = pl.multiple_of(bc * R, 8)

    @pl.loop(0, FPW)
    def _feat(fi):
        f = fh * FPW + fi
        pltpu.sync_copy(idxt_hbm.at[f, :, pl.ds(bag0, R)], idx_v)
        copies = [
            pltpu.async_copy(
                tab_hbm.at[idx_v.at[l, pl.ds(h * GROWS, GROWS)]],
                g_v.at[pl.ds(l * R + h * GROWS, GROWS)],
                sems.at[h],
            )
            for h in range(NWIN)
            for l in range(L)
        ]
        for h in range(NWIN):
            for cp in copies[h * L : (h + 1) * L]:
                cp.wait()

            @pl.loop(0, HBAGS)
            def _bag(b):
                base = h * GROWS + b
                acc = g_v[base, :]
                for l in range(1, L):
                    acc = acc + g_v[l * R + base, :]
                o_v[base, :] = acc

        pltpu.sync_copy(o_v, out_hbm.at[pl.ds(bag0, R), pl.ds(f * D, D)])


def kernel(indices, tables):
    packed, gidx = _tc_prep(
        tables.transpose(0, 2, 1), indices.transpose(0, 2, 1)
    )
    return _pooled_gather(packed.reshape(F * V, D), gidx)
